# baseline (speedup 1.0000x reference)
import jax
import jax.numpy as jnp
from jax.experimental import pallas as pl
from jax.experimental.pallas import tpu as pltpu

_NUM_CLASSES = 20
_STRIDE = 32.0
_INPUT_SIZE = 2560.0
_CONF_THRESH = 0.01
_G = 80
_N = _G * _G
_EPS = 1e-28
_B = 256
_NBLK = _N // _B


def _decode_kernel(pt_ref, grid_ref, boxes_ref, scores_ref, cls_ref, all8_ref):
    pt = pt_ref[...]
    conf = jax.nn.sigmoid(pt[0:1, :])
    logits = pt[1:1 + _NUM_CLASSES, :]
    m = jnp.max(logits, axis=0, keepdims=True)
    e = jnp.exp(logits - m)
    s = jnp.sum(e, axis=0, keepdims=True)
    probs = (e / s) * conf
    scores = jnp.max(probs, axis=0, keepdims=True)
    row_ids = jax.lax.broadcasted_iota(jnp.int32, probs.shape, 0)
    cls = jnp.min(jnp.where(probs == scores, row_ids, _NUM_CLASSES),
                  axis=0, keepdims=True)

    sig = jax.nn.sigmoid(pt[21:23, :])
    cxcy = (grid_ref[...] + sig) * _STRIDE
    wh = jnp.exp(pt[23:25, :])
    lo = jnp.clip((cxcy - wh * 0.5) / _INPUT_SIZE, 0.0, 1.0)
    hi = jnp.clip((cxcy + wh * 0.5) / _INPUT_SIZE, 0.0, 1.0)
    boxes = jnp.concatenate([lo, hi], axis=0)
    boxes_ref[...] = boxes
    scores_ref[...] = scores
    cls_ref[...] = cls
    clsf = cls.astype(jnp.float32)
    ob = boxes + clsf * 2.0
    area3 = ((boxes[2:3, :] - boxes[0:1, :])
             * (boxes[3:4, :] - boxes[1:2, :])) * (1.0 / 3.0)
    all8_ref[...] = jnp.concatenate([ob, area3], axis=0)


def _overlap(colc, x1r, y1r, x2r, y2r, ar):
    x1j = colc[:, 0:1]
    y1j = colc[:, 1:2]
    x2j = colc[:, 2:3]
    y2j = colc[:, 3:4]
    aj = colc[:, 4:5]
    w = jnp.maximum(_EPS, jnp.minimum(x2j, x2r) - jnp.maximum(x1j, x1r))
    h = jnp.maximum(_EPS, jnp.minimum(y2j, y2r) - jnp.maximum(y1j, y1r))
    inter = w * h
    return jnp.where(inter > aj + ar, 1.0, 0.0)


def _nms_kernel(rows_ref, start_ref, keep_ref):
    a_ids = jax.lax.broadcasted_iota(jnp.int32, (_B, _B), 0)
    b_ids = jax.lax.broadcasted_iota(jnp.int32, (_B, _B), 1)
    upper = a_ids < b_ids

    def blk_body(k, carry):
        koff = pl.multiple_of(k * _B, 128)
        cur = rows_ref[:, pl.ds(koff, _B)]
        x1r = cur[0:1, :]
        y1r = cur[1:2, :]
        x2r = cur[2:3, :]
        y2r = cur[3:4, :]
        ar = cur[4:5, :]

        def ext_body(c, cnt):
            off = pl.multiple_of(c * _B, 128)
            colc = jnp.swapaxes(rows_ref[:, pl.ds(off, _B)], 0, 1)
            ov = _overlap(colc, x1r, y1r, x2r, y2r, ar).astype(jnp.bfloat16)
            kb = keep_ref[0:1, pl.ds(off, _B)].astype(jnp.bfloat16)
            return cnt + jnp.dot(kb, ov, preferred_element_type=jnp.float32)

        ext_cnt = jax.lax.fori_loop(start_ref[k], k, ext_body,
                                    jnp.zeros((1, _B), jnp.float32))
        v = jnp.where(ext_cnt > 0.0, 0.0, cur[5:6, :])

        ovk = _overlap(jnp.swapaxes(cur, 0, 1), x1r, y1r, x2r, y2r, ar)
        M = jnp.where(upper, ovk, 0.0).astype(jnp.bfloat16)

        def cond(carry2):
            return carry2[1] > 0.0

        def body(carry2):
            keep, _ = carry2
            cnt = jnp.dot(keep.astype(jnp.bfloat16), M,
                          preferred_element_type=jnp.float32)
            new = jnp.where(cnt > 0.0, 0.0, v)
            diff = jnp.sum(jnp.abs(new - keep))
            return (new, diff)

        keep_f, _ = jax.lax.while_loop(cond, body, (v, jnp.float32(1.0)))

        keep_ref[0:1, pl.ds(koff, _B)] = keep_f
        return carry

    jax.lax.fori_loop(0, _NBLK, blk_body, jnp.int32(0))


def _decode(pt, grid):
    return pl.pallas_call(
        _decode_kernel,
        out_shape=(
            jax.ShapeDtypeStruct((4, _N), jnp.float32),
            jax.ShapeDtypeStruct((1, _N), jnp.float32),
            jax.ShapeDtypeStruct((1, _N), jnp.int32),
            jax.ShapeDtypeStruct((5, _N), jnp.float32),
        ),
    )(pt, grid)


def _nms(cur, ext_start):
    return pl.pallas_call(
        _nms_kernel,
        in_specs=[
            pl.BlockSpec((8, _N), lambda: (0, 0)),
            pl.BlockSpec(memory_space=pltpu.SMEM),
        ],
        out_specs=pl.BlockSpec((1, _N), lambda: (0, 0)),
        out_shape=jax.ShapeDtypeStruct((1, _N), jnp.float32),
    )(cur, ext_start)


def _make_grid_rows():
    gx, gy = jnp.meshgrid(jnp.arange(_G, dtype=jnp.float32),
                          jnp.arange(_G, dtype=jnp.float32), indexing='ij')
    return jnp.stack([gx, gy], axis=0).reshape(2, _N)


@jax.jit
def kernel(pred):
    pt = pred[0].T
    grid = _make_grid_rows()
    boxes_r, scores_r, cls_r, all8_r = _decode(pt, grid)

    scores = scores_r[0]
    cls = cls_r[0]

    sbits = jax.lax.bitcast_convert_type(scores, jnp.int32)
    off = jnp.int32(0x3C23D70A - 1)
    maxpart = jnp.int32(0x3F800000 - 0x3C23D70A + 1)
    part = jnp.clip(sbits - off, 0, maxpart)
    key = cls * jnp.int32(1 << 26) + (maxpart - part)
    key_s, order, x1s, y1s, x2s, y2s, a3s = jax.lax.sort(
        (key, jnp.arange(_N, dtype=jnp.int32),
         all8_r[0], all8_r[1], all8_r[2], all8_r[3], all8_r[4]),
        num_keys=1, is_stable=True)
    validf = jnp.where((key_s & jnp.int32((1 << 26) - 1)) < maxpart, 1.0, 0.0)
    cur = jnp.stack([x1s, y1s, x2s, y2s, a3s, validf, validf, validf],
                    axis=0)

    cls_s = jax.lax.shift_right_logical(key_s, 26)
    mincls = cls_s[::_B]
    maxcls = cls_s[_B - 1::_B]
    ext_start = jnp.sum((maxcls[None, :] < mincls[:, None]).astype(jnp.int32),
                        axis=1)

    keep_sorted = _nms(cur, ext_start)[0]
    _, keep_f = jax.lax.sort((order, keep_sorted), num_keys=1, is_stable=False)
    keep = keep_f > 0.5

    boxes = boxes_r.T
    return boxes, scores * keep, cls, keep


# speedup vs baseline: 1.2392x; 1.2392x over previous
import jax
import jax.numpy as jnp
from jax.experimental import pallas as pl
from jax.experimental.pallas import tpu as pltpu

_NUM_CLASSES = 20
_STRIDE = 32.0
_INPUT_SIZE = 2560.0
_CONF_THRESH = 0.01
_G = 80
_N = _G * _G
_EPS = 1e-28
_B = 640
_NBLK = _N // _B


def _decode_kernel(pt_ref, grid_ref, boxes_ref, scores_ref, cls_ref, all8_ref):
    pt = pt_ref[...]
    conf = jax.nn.sigmoid(pt[0:1, :])
    logits = pt[1:1 + _NUM_CLASSES, :]
    m = jnp.max(logits, axis=0, keepdims=True)
    e = jnp.exp(logits - m)
    s = jnp.sum(e, axis=0, keepdims=True)
    probs = (e / s) * conf
    scores = jnp.max(probs, axis=0, keepdims=True)
    row_ids = jax.lax.broadcasted_iota(jnp.int32, probs.shape, 0)
    cls = jnp.min(jnp.where(probs == scores, row_ids, _NUM_CLASSES),
                  axis=0, keepdims=True)

    sig = jax.nn.sigmoid(pt[21:23, :])
    cxcy = (grid_ref[...] + sig) * _STRIDE
    wh = jnp.exp(pt[23:25, :])
    lo = jnp.clip((cxcy - wh * 0.5) / _INPUT_SIZE, 0.0, 1.0)
    hi = jnp.clip((cxcy + wh * 0.5) / _INPUT_SIZE, 0.0, 1.0)
    boxes = jnp.concatenate([lo, hi], axis=0)
    boxes_ref[...] = boxes
    scores_ref[...] = scores
    cls_ref[...] = cls
    clsf = cls.astype(jnp.float32)
    ob = boxes + clsf * 2.0
    area3 = ((boxes[2:3, :] - boxes[0:1, :])
             * (boxes[3:4, :] - boxes[1:2, :])) * (1.0 / 3.0)
    all8_ref[...] = jnp.concatenate([ob, area3], axis=0)


def _overlap(colc, x1r, y1r, x2r, y2r, ar):
    x1j = colc[:, 0:1]
    y1j = colc[:, 1:2]
    x2j = colc[:, 2:3]
    y2j = colc[:, 3:4]
    aj = colc[:, 4:5]
    w = jnp.maximum(_EPS, jnp.minimum(x2j, x2r) - jnp.maximum(x1j, x1r))
    h = jnp.maximum(_EPS, jnp.minimum(y2j, y2r) - jnp.maximum(y1j, y1r))
    inter = w * h
    return jnp.where(inter > aj + ar, 1.0, 0.0)


def _nms_kernel(rows_ref, start_ref, keep_ref):
    a_ids = jax.lax.broadcasted_iota(jnp.int32, (_B, _B), 0)
    b_ids = jax.lax.broadcasted_iota(jnp.int32, (_B, _B), 1)
    upper = a_ids < b_ids

    def blk_body(k, carry):
        koff = pl.multiple_of(k * _B, 128)
        cur = rows_ref[:, pl.ds(koff, _B)]
        x1r = cur[0:1, :]
        y1r = cur[1:2, :]
        x2r = cur[2:3, :]
        y2r = cur[3:4, :]
        ar = cur[4:5, :]

        def ext_body(c, cnt):
            off = pl.multiple_of(c * _B, 128)
            colc = jnp.swapaxes(rows_ref[:, pl.ds(off, _B)], 0, 1)
            ov = _overlap(colc, x1r, y1r, x2r, y2r, ar).astype(jnp.bfloat16)
            kb = keep_ref[0:1, pl.ds(off, _B)].astype(jnp.bfloat16)
            return cnt + jnp.dot(kb, ov, preferred_element_type=jnp.float32)

        ext_cnt = jax.lax.fori_loop(start_ref[k], k, ext_body,
                                    jnp.zeros((1, _B), jnp.float32))
        v = jnp.where(ext_cnt > 0.0, 0.0, cur[5:6, :])

        ovk = _overlap(jnp.swapaxes(cur, 0, 1), x1r, y1r, x2r, y2r, ar)
        M = jnp.where(upper, ovk, 0.0).astype(jnp.bfloat16)

        def cond(carry2):
            return carry2[1] > 0.0

        def body(carry2):
            keep, _ = carry2
            cnt = jnp.dot(keep.astype(jnp.bfloat16), M,
                          preferred_element_type=jnp.float32)
            new = jnp.where(cnt > 0.0, 0.0, v)
            diff = jnp.sum(jnp.abs(new - keep))
            return (new, diff)

        keep_f, _ = jax.lax.while_loop(cond, body, (v, jnp.float32(1.0)))

        keep_ref[0:1, pl.ds(koff, _B)] = keep_f
        return carry

    jax.lax.fori_loop(0, _NBLK, blk_body, jnp.int32(0))


def _decode(pt, grid):
    return pl.pallas_call(
        _decode_kernel,
        out_shape=(
            jax.ShapeDtypeStruct((4, _N), jnp.float32),
            jax.ShapeDtypeStruct((1, _N), jnp.float32),
            jax.ShapeDtypeStruct((1, _N), jnp.int32),
            jax.ShapeDtypeStruct((5, _N), jnp.float32),
        ),
    )(pt, grid)


def _nms(cur, ext_start):
    return pl.pallas_call(
        _nms_kernel,
        in_specs=[
            pl.BlockSpec((8, _N), lambda: (0, 0)),
            pl.BlockSpec(memory_space=pltpu.SMEM),
        ],
        out_specs=pl.BlockSpec((1, _N), lambda: (0, 0)),
        out_shape=jax.ShapeDtypeStruct((1, _N), jnp.float32),
    )(cur, ext_start)


def _make_grid_rows():
    gx, gy = jnp.meshgrid(jnp.arange(_G, dtype=jnp.float32),
                          jnp.arange(_G, dtype=jnp.float32), indexing='ij')
    return jnp.stack([gx, gy], axis=0).reshape(2, _N)


@jax.jit
def kernel(pred):
    pt = pred[0].T
    grid = _make_grid_rows()
    boxes_r, scores_r, cls_r, all8_r = _decode(pt, grid)

    scores = scores_r[0]
    cls = cls_r[0]

    sbits = jax.lax.bitcast_convert_type(scores, jnp.int32)
    off = jnp.int32(0x3C23D70A - 1)
    maxpart = jnp.int32(0x3F800000 - 0x3C23D70A + 1)
    part = jnp.clip(sbits - off, 0, maxpart)
    key = cls * jnp.int32(1 << 26) + (maxpart - part)
    key_s, order, x1s, y1s, x2s, y2s, a3s = jax.lax.sort(
        (key, jnp.arange(_N, dtype=jnp.int32),
         all8_r[0], all8_r[1], all8_r[2], all8_r[3], all8_r[4]),
        num_keys=1, is_stable=True)
    validf = jnp.where((key_s & jnp.int32((1 << 26) - 1)) < maxpart, 1.0, 0.0)
    cur = jnp.stack([x1s, y1s, x2s, y2s, a3s, validf, validf, validf],
                    axis=0)

    cls_s = jax.lax.shift_right_logical(key_s, 26)
    mincls = cls_s[::_B]
    maxcls = cls_s[_B - 1::_B]
    ext_start = jnp.sum((maxcls[None, :] < mincls[:, None]).astype(jnp.int32),
                        axis=1)

    keep_sorted = _nms(cur, ext_start)[0]
    _, keep_f = jax.lax.sort((order, keep_sorted), num_keys=1, is_stable=False)
    keep = keep_f > 0.5

    boxes = boxes_r.T
    return boxes, scores * keep, cls, keep
